# Initial kernel scaffold; baseline (speedup 1.0000x reference)
import jax
import jax.numpy as jnp
from jax.experimental import pallas as pl


def kernel(x, batch, W1, b1, W2, b2):
    raise NotImplementedError("write your pallas kernel here")


# speedup vs baseline: 7.9563x; 7.9563x over previous
import functools

import jax
import jax.numpy as jnp
from jax.experimental import pallas as pl
from jax.experimental.pallas import tpu as pltpu


def _body(nblocks, S, xb, bb, W1, b1, W2, b2, U, out_ref, acc, meta):
    i = pl.program_id(0)

    @pl.when(i == 0)
    def _init():
        acc[...] = jnp.zeros_like(acc)
        meta[...] = jnp.zeros_like(meta)

    x = xb[...]
    h = jnp.tanh(
        jax.lax.dot_general(x, W1[...], (((1,), (0,)), ((), ())),
                            preferred_element_type=jnp.float32)
        + b1[...]
    )
    s = jnp.sum(h * W2[...], axis=1, keepdims=True) + b2[...]
    e = jnp.exp(s - U[...])
    w = x * e

    ids = bb[0]
    B = ids.shape[1]
    seg = jax.lax.broadcasted_iota(jnp.int32, (S, B), 0)
    oh = (seg == ids).astype(jnp.float32)

    acc[...] += jax.lax.dot_general(
        oh, w, (((1,), (0,)), ((), ())), preferred_element_type=jnp.float32)
    e2 = jnp.concatenate([e, jnp.ones_like(e)], axis=1)
    meta[...] += jax.lax.dot_general(
        oh, e2, (((1,), (0,)), ((), ())), preferred_element_type=jnp.float32)

    @pl.when(i == nblocks - 1)
    def _fin():
        denom = jnp.maximum(meta[:, 0:1], 1e-30)
        cnt = jnp.maximum(meta[:, 1:2], 1.0)
        out_ref[...] = acc[...] / (denom * cnt)


def kernel(x, batch, W1, b1, W2, b2):
    N, D = x.shape
    H = W1.shape[1]
    S = 512
    B = 1000
    assert N % B == 0
    nb = N // B

    batch3 = batch.reshape(nb, 1, B)
    U = (jnp.sum(jnp.abs(W2)) + b2[0]).reshape(1, 1)
    b1r = b1.reshape(1, H)
    b2r = b2.reshape(1, 1)
    W2r = W2.reshape(1, H)

    return pl.pallas_call(
        functools.partial(_body, nb, S),
        grid=(nb,),
        in_specs=[
            pl.BlockSpec((B, D), lambda i: (i, 0)),
            pl.BlockSpec((1, 1, B), lambda i: (i, 0, 0)),
            pl.BlockSpec((D, H), lambda i: (0, 0)),
            pl.BlockSpec((1, H), lambda i: (0, 0)),
            pl.BlockSpec((1, H), lambda i: (0, 0)),
            pl.BlockSpec((1, 1), lambda i: (0, 0)),
            pl.BlockSpec((1, 1), lambda i: (0, 0)),
        ],
        out_specs=pl.BlockSpec((S, D), lambda i: (0, 0)),
        out_shape=jax.ShapeDtypeStruct((S, D), jnp.float32),
        scratch_shapes=[
            pltpu.VMEM((S, D), jnp.float32),
            pltpu.VMEM((S, 2), jnp.float32),
        ],
    )(x, batch3, W1, b1r, W2r, b2r, U)
